# Initial kernel scaffold; baseline (speedup 1.0000x reference)
import jax
import jax.numpy as jnp
from jax.experimental import pallas as pl


def kernel(x, edge_index, W1, b1, W2, b2):
    raise NotImplementedError("write your pallas kernel here")


# speedup vs baseline: 9.2847x; 9.2847x over previous
import functools
import jax
import jax.numpy as jnp
from jax import lax
from jax.experimental import pallas as pl
from jax.experimental.pallas import tpu as pltpu
from jax.experimental.pallas import tpu_sc as plsc

N = 10000
E = 320000
D_IN = 128
WIDTH = 256

NC, NS, L = 2, 16, 16
NW = NC * NS
N_PAD = 10240
RPS = N_PAD // NS
EPW = E // NW
C = 80
NCHUNK = EPW // C

_mesh = lambda: plsc.VectorSubcoreMesh(core_axis_name="c", subcore_axis_name="s")


def _zero_vmem_2d(ref, rows, cols):
    def body(i, _):
        r = i // (cols // L)
        k = i % (cols // L)
        ref[r, pl.ds(k * L, L)] = jnp.zeros((L,), jnp.float32)
        return 0
    lax.fori_loop(0, rows * (cols // L), body, 0)


def _zero_vmem_1d(ref, n):
    def body(i, _):
        ref[pl.ds(i * L, L)] = jnp.zeros((L,), jnp.float32)
        return 0
    lax.fori_loop(0, n // L, body, 0)


@functools.partial(
    pl.kernel,
    out_type=jax.ShapeDtypeStruct((NC, N_PAD), jnp.float32),
    mesh=_mesh(),
    scratch_types=[
        pltpu.VMEM((C,), jnp.int32),
        pltpu.VMEM((C,), jnp.float32),
        pltpu.VMEM((RPS,), jnp.float32),
        pltpu.VMEM_SHARED((N_PAD,), jnp.float32),
    ],
)
def _deg_kernel(dst_hbm, out_hbm, idx_v, ones_v, slab_v, deg_sh):
    cid = lax.axis_index("c")
    sid = lax.axis_index("s")
    _zero_vmem_1d(slab_v, RPS)
    pltpu.sync_copy(slab_v, deg_sh.at[pl.ds(sid * RPS, RPS)])
    def fill(i, _):
        ones_v[pl.ds(i * L, L)] = jnp.ones((L,), jnp.float32)
        return 0
    lax.fori_loop(0, C // L, fill, 0)
    plsc.subcore_barrier()

    base0 = (cid * NS + sid) * EPW
    def body(i, _):
        pltpu.sync_copy(dst_hbm.at[pl.ds(base0 + i * C, C)], idx_v)
        pltpu.sync_copy(ones_v, deg_sh.at[idx_v], add=True)
        return 0
    lax.fori_loop(0, NCHUNK, body, 0)
    plsc.subcore_barrier()
    pltpu.sync_copy(deg_sh.at[pl.ds(sid * RPS, RPS)],
                    out_hbm.at[cid, pl.ds(sid * RPS, RPS)])


@functools.partial(
    pl.kernel,
    out_type=(
        jax.ShapeDtypeStruct((NC, N_PAD, D_IN), jnp.float32),
        jax.ShapeDtypeStruct((NC, N_PAD), jnp.float32),
    ),
    mesh=_mesh(),
    scratch_types=[
        pltpu.VMEM((C,), jnp.int32),
        pltpu.VMEM((C,), jnp.int32),
        pltpu.VMEM((C, D_IN), jnp.float32),
        pltpu.VMEM((C,), jnp.float32),
        pltpu.VMEM((64, D_IN), jnp.float32),
        pltpu.SemaphoreType.DMA,
        pltpu.SemaphoreType.DMA,
        pltpu.VMEM_SHARED((N_PAD, D_IN), jnp.float32),
        pltpu.VMEM_SHARED((N_PAD,), jnp.float32),
        pltpu.VMEM_SHARED((N_PAD,), jnp.float32),
    ],
)
def _edge_kernel(src_hbm, dst_hbm, m_hbm, norm_hbm, agg_out, c_out,
                 src_v, dst_v, rows_v, vals_v, slab_v, sem, sem2,
                 agg_sh, c_sh, norm_sh):
    cid = lax.axis_index("c")
    sid = lax.axis_index("s")

    _zero_vmem_2d(slab_v, 64, D_IN)
    def zbody(k, _):
        pltpu.sync_copy(slab_v, agg_sh.at[pl.ds(sid * RPS + k * 64, 64)])
        return 0
    lax.fori_loop(0, RPS // 64, zbody, 0)
    def zc(k, _):
        pltpu.sync_copy(slab_v.at[0], c_sh.at[pl.ds(sid * RPS + k * D_IN, D_IN)])
        return 0
    lax.fori_loop(0, RPS // D_IN, zc, 0)
    @pl.when(sid == 0)
    def _():
        pltpu.sync_copy(norm_hbm, norm_sh)
    plsc.subcore_barrier()

    base0 = (cid * NS + sid) * EPW
    def body(i, _):
        pltpu.sync_copy(src_hbm.at[pl.ds(base0 + i * C, C)], src_v)
        pltpu.sync_copy(dst_hbm.at[pl.ds(base0 + i * C, C)], dst_v)
        row_dma = pltpu.async_copy(m_hbm.at[src_v], rows_v, sem)
        nrm_dma = pltpu.async_copy(norm_sh.at[dst_v], vals_v, sem2)
        row_dma.wait()
        pltpu.sync_copy(rows_v, agg_sh.at[dst_v], add=True)
        nrm_dma.wait()
        pltpu.sync_copy(vals_v, c_sh.at[src_v], add=True)
        return 0
    lax.fori_loop(0, NCHUNK, body, 0)
    plsc.subcore_barrier()

    pltpu.sync_copy(agg_sh.at[pl.ds(sid * RPS, RPS)],
                    agg_out.at[cid, pl.ds(sid * RPS, RPS)])
    pltpu.sync_copy(c_sh.at[pl.ds(sid * RPS, RPS)],
                    c_out.at[cid, pl.ds(sid * RPS, RPS)])


def _tc1_body(d0_ref, d1_ref, x_ref, m_ref, norm_ref):
    deg = d0_ref[...] + d1_ref[...]
    nrm = lax.rsqrt(jnp.maximum(deg, 1.0))
    norm_ref[...] = nrm
    m_ref[...] = x_ref[...] * nrm


BN = 2048
G = N_PAD // BN

def _tc2_body(a0, a1, nrm, c0, c1, W1, b1, W2, b2, out_ref):
    i = pl.program_id(0)
    a = (a0[...] + a1[...]) * nrm[...]
    h = jnp.maximum(
        jnp.dot(a, W1[...], preferred_element_type=jnp.float32) + b1[...], 0.0)
    v = jnp.dot(h, W2[...], preferred_element_type=jnp.float32)
    part = jnp.sum(v * nrm[...] * (c0[...] + c1[...]))
    prev = jnp.where(i == 0, jnp.zeros((1, 1), jnp.float32), out_ref[...])
    acc = prev + part
    out_ref[...] = jnp.where(i == G - 1, acc / N + b2[...], acc)


def kernel(x, edge_index, W1, b1, W2, b2):
    src = edge_index[0].astype(jnp.int32)
    dst = edge_index[1].astype(jnp.int32)
    x_pad = jnp.pad(x, ((0, N_PAD - N), (0, 0)))

    deg_parts = _deg_kernel(dst)
    d0 = deg_parts[0].reshape(N_PAD, 1)
    d1 = deg_parts[1].reshape(N_PAD, 1)

    m, norm_col = pl.pallas_call(
        _tc1_body,
        out_shape=(
            jax.ShapeDtypeStruct((N_PAD, D_IN), jnp.float32),
            jax.ShapeDtypeStruct((N_PAD, 1), jnp.float32),
        ),
    )(d0, d1, x_pad)

    agg_parts, c_parts = _edge_kernel(src, dst, m, norm_col.reshape(N_PAD))

    blk = lambda *s: pl.BlockSpec(s, lambda i: (0,) * len(s))
    out2d = pl.pallas_call(
        _tc2_body,
        grid=(G,),
        in_specs=[
            pl.BlockSpec((BN, D_IN), lambda i: (i, 0)),
            pl.BlockSpec((BN, D_IN), lambda i: (i, 0)),
            pl.BlockSpec((BN, 1), lambda i: (i, 0)),
            pl.BlockSpec((BN, 1), lambda i: (i, 0)),
            pl.BlockSpec((BN, 1), lambda i: (i, 0)),
            blk(D_IN, WIDTH), blk(1, WIDTH), blk(WIDTH, 1), blk(1, 1),
        ],
        out_specs=pl.BlockSpec((1, 1), lambda i: (0, 0)),
        out_shape=jax.ShapeDtypeStruct((1, 1), jnp.float32),
    )(agg_parts[0], agg_parts[1], norm_col,
      c_parts[0].reshape(N_PAD, 1), c_parts[1].reshape(N_PAD, 1),
      W1, b1.reshape(1, WIDTH), W2, b2.reshape(1, 1))

    return out2d[0, 0]
